# Initial kernel scaffold; baseline (speedup 1.0000x reference)
import jax
import jax.numpy as jnp
from jax.experimental import pallas as pl


def kernel(item_index, user_index, item_emb, user_emb, W0, b0, W1, b1, W2, b2, W3, b3):
    raise NotImplementedError("write your pallas kernel here")


# speedup vs baseline: 2.6069x; 2.6069x over previous
import functools

import jax
import jax.numpy as jnp
from jax import lax
from jax.experimental import pallas as pl
from jax.experimental.pallas import tpu as pltpu
from jax.experimental.pallas import tpu_sc as plsc

BATCH = 16384
DIM = 128

_NC = 2
_NS = 16
_NW = _NC * _NS
_B_PER_W = BATCH // _NW
_CHUNK = 128
_NCHUNK = _B_PER_W // _CHUNK


def _sc_gather_body(uidx_hbm, iidx_hbm, user_emb_hbm, item_emb_hbm,
                    ue_hbm, ie_hbm,
                    uidx_v, iidx_v, urows_v, irows_v, usem, isem):
    wid = lax.axis_index("s") * _NC + lax.axis_index("c")
    base = wid * _B_PER_W
    pltpu.sync_copy(uidx_hbm.at[pl.ds(wid * _NCHUNK, _NCHUNK)], uidx_v)
    pltpu.sync_copy(iidx_hbm.at[pl.ds(wid * _NCHUNK, _NCHUNK)], iidx_v)
    for c in range(_NCHUNK):
        cu = pltpu.async_copy(user_emb_hbm.at[uidx_v.at[c]], urows_v, usem)
        ci = pltpu.async_copy(item_emb_hbm.at[iidx_v.at[c]], irows_v, isem)
        cu.wait()
        pltpu.sync_copy(urows_v, ue_hbm.at[pl.ds(base + c * _CHUNK, _CHUNK)])
        ci.wait()
        pltpu.sync_copy(irows_v, ie_hbm.at[pl.ds(base + c * _CHUNK, _CHUNK)])


def _sc_gather(uidx, iidx, user_emb, item_emb):
    mesh = plsc.VectorSubcoreMesh(core_axis_name="c", subcore_axis_name="s")
    return pl.kernel(
        _sc_gather_body,
        out_type=(
            jax.ShapeDtypeStruct((BATCH, DIM), jnp.float32),
            jax.ShapeDtypeStruct((BATCH, DIM), jnp.float32),
        ),
        mesh=mesh,
        scratch_types=[
            pltpu.VMEM((_NCHUNK, _CHUNK), jnp.int32),
            pltpu.VMEM((_NCHUNK, _CHUNK), jnp.int32),
            pltpu.VMEM((_CHUNK, DIM), jnp.float32),
            pltpu.VMEM((_CHUNK, DIM), jnp.float32),
            pltpu.SemaphoreType.DMA,
            pltpu.SemaphoreType.DMA,
        ],
    )(uidx, iidx, user_emb, item_emb)


_TILE = 2048


def _mlp_body(ue_ref, ie_ref, w0u_ref, w0i_ref, b0_ref, w1_ref, b1_ref,
              w2_ref, b2_ref, w3_ref, b3_ref, out_ref):
    f32 = jnp.float32
    h = jnp.dot(ue_ref[...], w0u_ref[...], preferred_element_type=f32)
    h += jnp.dot(ie_ref[...], w0i_ref[...], preferred_element_type=f32)
    h = jnp.maximum(h + b0_ref[...], 0.0)
    h = jnp.dot(h, w1_ref[...], preferred_element_type=f32)
    h = jnp.maximum(h + b1_ref[...], 0.0)
    h = jnp.dot(h, w2_ref[...], preferred_element_type=f32)
    h = jnp.maximum(h + b2_ref[...], 0.0)
    h = jnp.dot(h, w3_ref[...], preferred_element_type=f32)
    h = jnp.maximum(h + b3_ref[...], 0.0)
    out_ref[...] = 1.0 / (1.0 + jnp.exp(-h))


def _mlp(ue, ie, w0u, w0i, b0, w1, b1, w2, b2, w3, b3):
    grid = (BATCH // _TILE,)
    full = lambda shape: pl.BlockSpec(shape, lambda i: (0, 0))
    return pl.pallas_call(
        _mlp_body,
        grid=grid,
        in_specs=[
            pl.BlockSpec((_TILE, DIM), lambda i: (i, 0)),
            pl.BlockSpec((_TILE, DIM), lambda i: (i, 0)),
            full(w0u.shape), full(w0i.shape), full(b0.shape),
            full(w1.shape), full(b1.shape),
            full(w2.shape), full(b2.shape),
            full(w3.shape), full(b3.shape),
        ],
        out_specs=pl.BlockSpec((_TILE, 1), lambda i: (i, 0)),
        out_shape=jax.ShapeDtypeStruct((BATCH, 1), jnp.float32),
    )(ue, ie, w0u, w0i, b0, w1, b1, w2, b2, w3, b3)


def kernel(item_index, user_index, item_emb, user_emb,
           W0, b0, W1, b1, W2, b2, W3, b3):
    uidx = user_index.astype(jnp.int32).reshape(BATCH // _CHUNK, _CHUNK)
    iidx = item_index.astype(jnp.int32).reshape(BATCH // _CHUNK, _CHUNK)
    ue, ie = _sc_gather(uidx, iidx, user_emb, item_emb)

    w0u = W0[:, :DIM].T
    w0i = W0[:, DIM:].T
    out = _mlp(ue, ie,
               w0u, w0i, b0.reshape(1, -1),
               W1.T, b1.reshape(1, -1),
               W2.T, b2.reshape(1, -1),
               W3.T, b3.reshape(1, -1))
    return out


# speedup vs baseline: 2.6634x; 1.0217x over previous
import functools

import jax
import jax.numpy as jnp
from jax import lax
from jax.experimental import pallas as pl
from jax.experimental.pallas import tpu as pltpu
from jax.experimental.pallas import tpu_sc as plsc

BATCH = 16384
DIM = 128

_NC = 2
_NS = 16
_NW = _NC * _NS
_B_PER_W = BATCH // _NW
_CHUNK = 128
_NCHUNK = _B_PER_W // _CHUNK


def _sc_gather_body(uidx_hbm, iidx_hbm, user_emb_hbm, item_emb_hbm,
                    ue_hbm, ie_hbm,
                    uidx_v, iidx_v,
                    ur0, ur1, ir0, ir1,
                    g0, g1, s0, s1):
    wid = lax.axis_index("s") * _NC + lax.axis_index("c")
    base = wid * _B_PER_W
    ubuf, ibuf = (ur0, ur1), (ir0, ir1)
    gsem, ssem = (g0, g1), (s0, s1)
    pltpu.sync_copy(uidx_hbm.at[pl.ds(wid * _NCHUNK, _NCHUNK)], uidx_v)
    pltpu.sync_copy(iidx_hbm.at[pl.ds(wid * _NCHUNK, _NCHUNK)], iidx_v)

    def fire_gather(c, s):
        cu = pltpu.async_copy(user_emb_hbm.at[uidx_v.at[c]], ubuf[s], gsem[s])
        ci = pltpu.async_copy(item_emb_hbm.at[iidx_v.at[c]], ibuf[s], gsem[s])
        return cu, ci

    scat = [None, None]
    gath = [None, None]
    gath[0] = fire_gather(0, 0)
    for c in range(_NCHUNK):
        s = c % 2
        if c + 1 < _NCHUNK:
            if scat[1 - s] is not None:
                scat[1 - s][0].wait()
                scat[1 - s][1].wait()
            gath[1 - s] = fire_gather(c + 1, 1 - s)
        gath[s][0].wait()
        gath[s][1].wait()
        off = base + c * _CHUNK
        su = pltpu.async_copy(ubuf[s], ue_hbm.at[pl.ds(off, _CHUNK)], ssem[s])
        si = pltpu.async_copy(ibuf[s], ie_hbm.at[pl.ds(off, _CHUNK)], ssem[s])
        scat[s] = (su, si)
    for s in range(2):
        if scat[s] is not None:
            scat[s][0].wait()
            scat[s][1].wait()


def _sc_gather(uidx, iidx, user_emb, item_emb):
    mesh = plsc.VectorSubcoreMesh(core_axis_name="c", subcore_axis_name="s")
    return pl.kernel(
        _sc_gather_body,
        out_type=(
            jax.ShapeDtypeStruct((BATCH, DIM), jnp.float32),
            jax.ShapeDtypeStruct((BATCH, DIM), jnp.float32),
        ),
        mesh=mesh,
        scratch_types=[
            pltpu.VMEM((_NCHUNK, _CHUNK), jnp.int32),
            pltpu.VMEM((_NCHUNK, _CHUNK), jnp.int32),
            pltpu.VMEM((_CHUNK, DIM), jnp.float32),
            pltpu.VMEM((_CHUNK, DIM), jnp.float32),
            pltpu.VMEM((_CHUNK, DIM), jnp.float32),
            pltpu.VMEM((_CHUNK, DIM), jnp.float32),
            pltpu.SemaphoreType.DMA,
            pltpu.SemaphoreType.DMA,
            pltpu.SemaphoreType.DMA,
            pltpu.SemaphoreType.DMA,
        ],
    )(uidx, iidx, user_emb, item_emb)


_TILE = 2048


def _mlp_body(ue_ref, ie_ref, w0u_ref, w0i_ref, b0_ref, w1_ref, b1_ref,
              w2_ref, b2_ref, w3_ref, b3_ref, out_ref):
    f32 = jnp.float32
    h = jnp.dot(ue_ref[...], w0u_ref[...], preferred_element_type=f32)
    h += jnp.dot(ie_ref[...], w0i_ref[...], preferred_element_type=f32)
    h = jnp.maximum(h + b0_ref[...], 0.0)
    h = jnp.dot(h, w1_ref[...], preferred_element_type=f32)
    h = jnp.maximum(h + b1_ref[...], 0.0)
    h = jnp.dot(h, w2_ref[...], preferred_element_type=f32)
    h = jnp.maximum(h + b2_ref[...], 0.0)
    h = jnp.dot(h, w3_ref[...], preferred_element_type=f32)
    h = jnp.maximum(h + b3_ref[...], 0.0)
    out_ref[...] = 1.0 / (1.0 + jnp.exp(-h))


def _mlp(ue, ie, w0u, w0i, b0, w1, b1, w2, b2, w3, b3):
    grid = (BATCH // _TILE,)
    full = lambda shape: pl.BlockSpec(shape, lambda i: (0, 0))
    return pl.pallas_call(
        _mlp_body,
        grid=grid,
        in_specs=[
            pl.BlockSpec((_TILE, DIM), lambda i: (i, 0)),
            pl.BlockSpec((_TILE, DIM), lambda i: (i, 0)),
            full(w0u.shape), full(w0i.shape), full(b0.shape),
            full(w1.shape), full(b1.shape),
            full(w2.shape), full(b2.shape),
            full(w3.shape), full(b3.shape),
        ],
        out_specs=pl.BlockSpec((_TILE, 1), lambda i: (i, 0)),
        out_shape=jax.ShapeDtypeStruct((BATCH, 1), jnp.float32),
    )(ue, ie, w0u, w0i, b0, w1, b1, w2, b2, w3, b3)


def kernel(item_index, user_index, item_emb, user_emb,
           W0, b0, W1, b1, W2, b2, W3, b3):
    uidx = user_index.astype(jnp.int32).reshape(BATCH // _CHUNK, _CHUNK)
    iidx = item_index.astype(jnp.int32).reshape(BATCH // _CHUNK, _CHUNK)
    ue, ie = _sc_gather(uidx, iidx, user_emb, item_emb)

    w0u = W0[:, :DIM].T
    w0i = W0[:, DIM:].T
    out = _mlp(ue, ie,
               w0u, w0i, b0.reshape(1, -1),
               W1.T, b1.reshape(1, -1),
               W2.T, b2.reshape(1, -1),
               W3.T, b3.reshape(1, -1))
    return out


# speedup vs baseline: 2.7592x; 1.0360x over previous
import jax
import jax.numpy as jnp
from jax import lax
from jax.experimental import pallas as pl
from jax.experimental.pallas import tpu as pltpu
from jax.experimental.pallas import tpu_sc as plsc

BATCH = 16384
DIM = 128

_NC = 2
_NS = 16
_NW = _NC * _NS
_CHUNK = 128


def _make_sc_gather_body(nchunk):
    b_per_w = nchunk * _CHUNK

    def body(uidx_hbm, iidx_hbm, user_emb_hbm, item_emb_hbm,
             ue_hbm, ie_hbm,
             uidx_v, iidx_v,
             ur0, ur1, ir0, ir1,
             g0, g1, s0, s1):
        wid = lax.axis_index("s") * _NC + lax.axis_index("c")
        base = wid * b_per_w
        ubuf, ibuf = (ur0, ur1), (ir0, ir1)
        gsem, ssem = (g0, g1), (s0, s1)
        pltpu.sync_copy(uidx_hbm.at[pl.ds(wid * nchunk, nchunk)], uidx_v)
        pltpu.sync_copy(iidx_hbm.at[pl.ds(wid * nchunk, nchunk)], iidx_v)

        def fire_gather(c, s):
            cu = pltpu.async_copy(user_emb_hbm.at[uidx_v.at[c]], ubuf[s], gsem[s])
            ci = pltpu.async_copy(item_emb_hbm.at[iidx_v.at[c]], ibuf[s], gsem[s])
            return cu, ci

        scat = [None, None]
        gath = [None, None]
        gath[0] = fire_gather(0, 0)
        for c in range(nchunk):
            s = c % 2
            if c + 1 < nchunk:
                if scat[1 - s] is not None:
                    scat[1 - s][0].wait()
                    scat[1 - s][1].wait()
                gath[1 - s] = fire_gather(c + 1, 1 - s)
            gath[s][0].wait()
            gath[s][1].wait()
            off = base + c * _CHUNK
            su = pltpu.async_copy(ubuf[s], ue_hbm.at[pl.ds(off, _CHUNK)], ssem[s])
            si = pltpu.async_copy(ibuf[s], ie_hbm.at[pl.ds(off, _CHUNK)], ssem[s])
            scat[s] = (su, si)
        for s in range(2):
            if scat[s] is not None:
                scat[s][0].wait()
                scat[s][1].wait()

    return body


def _sc_gather(uidx, iidx, user_emb, item_emb):
    rows = uidx.shape[0] * _CHUNK
    nchunk = rows // (_NW * _CHUNK)
    mesh = plsc.VectorSubcoreMesh(core_axis_name="c", subcore_axis_name="s")
    return pl.kernel(
        _make_sc_gather_body(nchunk),
        out_type=(
            jax.ShapeDtypeStruct((rows, DIM), jnp.float32),
            jax.ShapeDtypeStruct((rows, DIM), jnp.float32),
        ),
        mesh=mesh,
        scratch_types=[
            pltpu.VMEM((nchunk, _CHUNK), jnp.int32),
            pltpu.VMEM((nchunk, _CHUNK), jnp.int32),
            pltpu.VMEM((_CHUNK, DIM), jnp.float32),
            pltpu.VMEM((_CHUNK, DIM), jnp.float32),
            pltpu.VMEM((_CHUNK, DIM), jnp.float32),
            pltpu.VMEM((_CHUNK, DIM), jnp.float32),
            pltpu.SemaphoreType.DMA,
            pltpu.SemaphoreType.DMA,
            pltpu.SemaphoreType.DMA,
            pltpu.SemaphoreType.DMA,
        ],
    )(uidx, iidx, user_emb, item_emb)


_TILE = 2048


def _mlp_body(ue_ref, ie_ref, w0u_ref, w0i_ref, b0_ref, w1_ref, b1_ref,
              w2_ref, b2_ref, w3_ref, b3_ref, out_ref):
    f32 = jnp.float32
    h = jnp.dot(ue_ref[...], w0u_ref[...], preferred_element_type=f32)
    h += jnp.dot(ie_ref[...], w0i_ref[...], preferred_element_type=f32)
    h = jnp.maximum(h + b0_ref[...], 0.0)
    h = jnp.dot(h, w1_ref[...], preferred_element_type=f32)
    h = jnp.maximum(h + b1_ref[...], 0.0)
    h = jnp.dot(h, w2_ref[...], preferred_element_type=f32)
    h = jnp.maximum(h + b2_ref[...], 0.0)
    h = jnp.dot(h, w3_ref[...], preferred_element_type=f32)
    h = jnp.maximum(h + b3_ref[...], 0.0)
    out_ref[...] = 1.0 / (1.0 + jnp.exp(-h))


def _mlp(ue, ie, w0u, w0i, b0, w1, b1, w2, b2, w3, b3):
    rows = ue.shape[0]
    grid = (rows // _TILE,)
    full = lambda shape: pl.BlockSpec(shape, lambda i: (0, 0))
    return pl.pallas_call(
        _mlp_body,
        grid=grid,
        in_specs=[
            pl.BlockSpec((_TILE, DIM), lambda i: (i, 0)),
            pl.BlockSpec((_TILE, DIM), lambda i: (i, 0)),
            full(w0u.shape), full(w0i.shape), full(b0.shape),
            full(w1.shape), full(b1.shape),
            full(w2.shape), full(b2.shape),
            full(w3.shape), full(b3.shape),
        ],
        out_specs=pl.BlockSpec((_TILE, 1), lambda i: (i, 0)),
        out_shape=jax.ShapeDtypeStruct((rows, 1), jnp.float32),
    )(ue, ie, w0u, w0i, b0, w1, b1, w2, b2, w3, b3)


_NSPLIT = 2


def kernel(item_index, user_index, item_emb, user_emb,
           W0, b0, W1, b1, W2, b2, W3, b3):
    uidx = user_index.astype(jnp.int32).reshape(BATCH // _CHUNK, _CHUNK)
    iidx = item_index.astype(jnp.int32).reshape(BATCH // _CHUNK, _CHUNK)

    w0u = W0[:, :DIM].T
    w0i = W0[:, DIM:].T
    wargs = (w0u, w0i, b0.reshape(1, -1),
             W1.T, b1.reshape(1, -1),
             W2.T, b2.reshape(1, -1),
             W3.T, b3.reshape(1, -1))

    srows = (BATCH // _CHUNK) // _NSPLIT
    gathered = []
    for h in range(_NSPLIT):
        uh = lax.slice_in_dim(uidx, h * srows, (h + 1) * srows, axis=0)
        ih = lax.slice_in_dim(iidx, h * srows, (h + 1) * srows, axis=0)
        gathered.append(_sc_gather(uh, ih, user_emb, item_emb))
    outs = [_mlp(ue, ie, *wargs) for ue, ie in gathered]
    if _NSPLIT == 1:
        return outs[0]
    return jnp.concatenate(outs, axis=0)


# speedup vs baseline: 2.7633x; 1.0015x over previous
import jax
import jax.numpy as jnp
from jax import lax
from jax.experimental import pallas as pl
from jax.experimental.pallas import tpu as pltpu
from jax.experimental.pallas import tpu_sc as plsc

BATCH = 16384
DIM = 128

_NC = 2
_NS = 16
_NW = _NC * _NS
_CHUNK = 128


def _make_sc_gather_body(nchunk):
    b_per_w = nchunk * _CHUNK

    def body(uidx_hbm, iidx_hbm, user_emb_hbm, item_emb_hbm,
             ue_hbm, ie_hbm,
             uidx_v, iidx_v,
             ur0, ur1, ir0, ir1,
             g0, g1, s0, s1):
        wid = lax.axis_index("s") * _NC + lax.axis_index("c")
        base = wid * b_per_w
        ubuf, ibuf = (ur0, ur1), (ir0, ir1)
        gsem, ssem = (g0, g1), (s0, s1)
        pltpu.sync_copy(uidx_hbm.at[pl.ds(wid * nchunk, nchunk)], uidx_v)
        pltpu.sync_copy(iidx_hbm.at[pl.ds(wid * nchunk, nchunk)], iidx_v)

        def fire_gather(c, s):
            cu = pltpu.async_copy(user_emb_hbm.at[uidx_v.at[c]], ubuf[s], gsem[s])
            ci = pltpu.async_copy(item_emb_hbm.at[iidx_v.at[c]], ibuf[s], gsem[s])
            return cu, ci

        scat = [None, None]
        gath = [None, None]
        gath[0] = fire_gather(0, 0)
        for c in range(nchunk):
            s = c % 2
            if c + 1 < nchunk:
                if scat[1 - s] is not None:
                    scat[1 - s][0].wait()
                    scat[1 - s][1].wait()
                gath[1 - s] = fire_gather(c + 1, 1 - s)
            gath[s][0].wait()
            gath[s][1].wait()
            off = base + c * _CHUNK
            su = pltpu.async_copy(ubuf[s], ue_hbm.at[pl.ds(off, _CHUNK)], ssem[s])
            si = pltpu.async_copy(ibuf[s], ie_hbm.at[pl.ds(off, _CHUNK)], ssem[s])
            scat[s] = (su, si)
        for s in range(2):
            if scat[s] is not None:
                scat[s][0].wait()
                scat[s][1].wait()

    return body


def _sc_gather(uidx, iidx, user_emb, item_emb):
    rows = uidx.shape[0] * _CHUNK
    nchunk = rows // (_NW * _CHUNK)
    mesh = plsc.VectorSubcoreMesh(core_axis_name="c", subcore_axis_name="s")
    return pl.kernel(
        _make_sc_gather_body(nchunk),
        out_type=(
            jax.ShapeDtypeStruct((rows, DIM), jnp.float32),
            jax.ShapeDtypeStruct((rows, DIM), jnp.float32),
        ),
        mesh=mesh,
        scratch_types=[
            pltpu.VMEM((nchunk, _CHUNK), jnp.int32),
            pltpu.VMEM((nchunk, _CHUNK), jnp.int32),
            pltpu.VMEM((_CHUNK, DIM), jnp.float32),
            pltpu.VMEM((_CHUNK, DIM), jnp.float32),
            pltpu.VMEM((_CHUNK, DIM), jnp.float32),
            pltpu.VMEM((_CHUNK, DIM), jnp.float32),
            pltpu.SemaphoreType.DMA,
            pltpu.SemaphoreType.DMA,
            pltpu.SemaphoreType.DMA,
            pltpu.SemaphoreType.DMA,
        ],
    )(uidx, iidx, user_emb, item_emb)


_TILE = 2048


def _dot_t(x, w):
    return lax.dot_general(x, w, (((1,), (1,)), ((), ())),
                           preferred_element_type=jnp.float32)


def _mlp_body(ue_ref, ie_ref, w0u_ref, w0i_ref, b0_ref, w1_ref, b1_ref,
              w2_ref, b2_ref, w3_ref, b3_ref, out_ref):
    h = _dot_t(ue_ref[...], w0u_ref[...]) + _dot_t(ie_ref[...], w0i_ref[...])
    h = jnp.maximum(h + b0_ref[...], 0.0)
    h = jnp.maximum(_dot_t(h, w1_ref[...]) + b1_ref[...], 0.0)
    h = jnp.maximum(_dot_t(h, w2_ref[...]) + b2_ref[...], 0.0)
    h = jnp.dot(h, w3_ref[...], preferred_element_type=jnp.float32)
    h = jnp.maximum(h + b3_ref[...], 0.0)
    out_ref[...] = 1.0 / (1.0 + jnp.exp(-h))


def _mlp(ue, ie, w0u, w0i, b0, w1, b1, w2, b2, w3, b3):
    rows = ue.shape[0]
    grid = (rows // _TILE,)
    full = lambda shape: pl.BlockSpec(shape, lambda i: (0, 0))
    return pl.pallas_call(
        _mlp_body,
        grid=grid,
        in_specs=[
            pl.BlockSpec((_TILE, DIM), lambda i: (i, 0)),
            pl.BlockSpec((_TILE, DIM), lambda i: (i, 0)),
            full(w0u.shape), full(w0i.shape), full(b0.shape),
            full(w1.shape), full(b1.shape),
            full(w2.shape), full(b2.shape),
            full(w3.shape), full(b3.shape),
        ],
        out_specs=pl.BlockSpec((_TILE, 1), lambda i: (i, 0)),
        out_shape=jax.ShapeDtypeStruct((rows, 1), jnp.float32),
    )(ue, ie, w0u, w0i, b0, w1, b1, w2, b2, w3, b3)


_NSPLIT = 2


def kernel(item_index, user_index, item_emb, user_emb,
           W0, b0, W1, b1, W2, b2, W3, b3):
    uidx = user_index.astype(jnp.int32).reshape(BATCH // _CHUNK, _CHUNK)
    iidx = item_index.astype(jnp.int32).reshape(BATCH // _CHUNK, _CHUNK)

    w0u = W0[:, :DIM]
    w0i = W0[:, DIM:]
    wargs = (w0u, w0i, b0.reshape(1, -1),
             W1, b1.reshape(1, -1),
             W2, b2.reshape(1, -1),
             W3.T, b3.reshape(1, -1))

    srows = (BATCH // _CHUNK) // _NSPLIT
    gathered = []
    for h in range(_NSPLIT):
        uh = lax.slice_in_dim(uidx, h * srows, (h + 1) * srows, axis=0)
        ih = lax.slice_in_dim(iidx, h * srows, (h + 1) * srows, axis=0)
        gathered.append(_sc_gather(uh, ih, user_emb, item_emb))
    outs = [_mlp(ue, ie, *wargs) for ue, ie in gathered]
    if _NSPLIT == 1:
        return outs[0]
    return jnp.concatenate(outs, axis=0)
